# baseline (speedup 1.0000x reference)
import functools

import jax
import jax.numpy as jnp
from jax import lax
from jax.experimental import pallas as pl
from jax.experimental.pallas import tpu as pltpu
from jax.experimental.pallas import tpu_sc as plsc

_B = 16384
_D = 64
_NC = 2
_NS = 16
_NW = _NC * _NS
_BPW = _B // _NW
_NCHUNK = 4
_CH = _BPW // _NCHUNK
_LANES = 16
_ROWCH = _D // _LANES


def _cl_body(feat_hbm, lab_hbm, cent_hbm, out_hbm, idx_v, rows_v, feat_v,
             acc_v, sem):
    c = lax.axis_index("c")
    s = lax.axis_index("s")
    wid = s * _NC + c
    base = wid * _BPW

    pltpu.sync_copy(lab_hbm.at[wid], idx_v)
    pltpu.sync_copy(feat_hbm.at[pl.ds(base, _BPW)], feat_v)

    copies = []
    for j in range(_NCHUNK):
        copies.append(
            pltpu.async_copy(cent_hbm.at[idx_v.at[j]],
                             rows_v.at[pl.ds(j * _CH, _CH)], sem))
    for cp in copies:
        cp.wait()

    def row_step(i, acc):
        for k in range(_ROWCH):
            f = feat_v[i, pl.ds(k * _LANES, _LANES)]
            g = rows_v[i, pl.ds(k * _LANES, _LANES)]
            d = f - g
            acc = acc + d * d
        return acc

    acc = lax.fori_loop(0, _BPW, row_step, jnp.zeros((_LANES,), jnp.float32))
    acc_v[...] = acc * (1.0 / _B)
    pltpu.sync_copy(acc_v, out_hbm.at[wid])


@functools.partial(jax.jit, donate_argnums=())
def kernel(features, labels, centers):
    labels_r = labels.astype(jnp.int32).reshape(_NW, _NCHUNK, _CH)
    mesh = plsc.VectorSubcoreMesh(core_axis_name="c", subcore_axis_name="s")
    partials = pl.kernel(
        _cl_body,
        out_type=jax.ShapeDtypeStruct((_NW, _LANES), jnp.float32),
        mesh=mesh,
        scratch_types=[
            pltpu.VMEM((_NCHUNK, _CH), jnp.int32),
            pltpu.VMEM((_BPW, _D), jnp.float32),
            pltpu.VMEM((_BPW, _D), jnp.float32),
            pltpu.VMEM((_LANES,), jnp.float32),
            pltpu.SemaphoreType.DMA,
        ],
        compiler_params=pltpu.CompilerParams(use_tc_tiling_on_sc=False),
    )(features, labels_r, centers)
    return jnp.sum(partials)


# speedup vs baseline: 1.7665x; 1.7665x over previous
import functools

import jax
import jax.numpy as jnp
from jax import lax
from jax.experimental import pallas as pl
from jax.experimental.pallas import tpu as pltpu
from jax.experimental.pallas import tpu_sc as plsc

_B = 16384
_D = 64
_V = 100000
_NC = 2
_NS = 16
_NW = _NC * _NS
_ROUNDS = _D // _NW
_LANES = 16
_FCH = 4096
_NFCH = _B // _FCH


def _cl_body(feat_hbm, lab_hbm, cent_hbm, out_hbm, row_v, feat_v, lab_v,
             acc_v, sem):
    c = lax.axis_index("c")
    s = lax.axis_index("s")
    wid = s * _NC + c

    pltpu.sync_copy(lab_hbm, lab_v)

    acc = jnp.zeros((_LANES,), jnp.float32)
    for r in range(_ROUNDS):
        comp = r * _NW + wid
        pltpu.sync_copy(cent_hbm.at[comp], row_v)
        for fc in range(_NFCH):
            pltpu.sync_copy(feat_hbm.at[comp, pl.ds(fc * _FCH, _FCH)], feat_v)

            def chunk_step(i, acc, fc=fc):
                idx = lab_v[pl.ds(fc * _FCH + i * _LANES, _LANES)]
                g = plsc.load_gather(row_v, [idx])
                f = feat_v[pl.ds(i * _LANES, _LANES)]
                d = f - g
                return acc + d * d

            acc = lax.fori_loop(0, _FCH // _LANES, chunk_step, acc)

    acc_v[...] = acc * (1.0 / _B)
    pltpu.sync_copy(acc_v, out_hbm.at[wid])


@jax.jit
def kernel(features, labels, centers):
    labels_i = labels.astype(jnp.int32)
    feat_t = features.T
    cent_t = centers.T
    mesh = plsc.VectorSubcoreMesh(core_axis_name="c", subcore_axis_name="s")
    partials = pl.kernel(
        _cl_body,
        out_type=jax.ShapeDtypeStruct((_NW, _LANES), jnp.float32),
        mesh=mesh,
        scratch_types=[
            pltpu.VMEM((_V,), jnp.float32),
            pltpu.VMEM((_FCH,), jnp.float32),
            pltpu.VMEM((_B,), jnp.int32),
            pltpu.VMEM((_LANES,), jnp.float32),
            pltpu.SemaphoreType.DMA,
        ],
        compiler_params=pltpu.CompilerParams(use_tc_tiling_on_sc=True,
                                            needs_layout_passes=False),
    )(feat_t, labels_i, cent_t)
    return jnp.sum(partials)


# speedup vs baseline: 2.0531x; 1.1622x over previous
import functools

import jax
import jax.numpy as jnp
from jax import lax
from jax.experimental import pallas as pl
from jax.experimental.pallas import tpu as pltpu
from jax.experimental.pallas import tpu_sc as plsc

_B = 16384
_D = 64
_V = 100000
_NC = 2
_NS = 16
_NW = _NC * _NS
_ROUNDS = _D // _NW
_LANES = 16
_FCH = 4096
_NFCH = _B // _FCH


def _cl_body(feat_hbm, lab_hbm, cent_hbm, out_hbm, row_v, feat_v, lab_v,
             acc_v, sem):
    c = lax.axis_index("c")
    s = lax.axis_index("s")
    wid = s * _NC + c

    pltpu.sync_copy(lab_hbm, lab_v)

    zero = jnp.zeros((_LANES,), jnp.float32)
    accs = (zero, zero, zero, zero)
    for r in range(_ROUNDS):
        comp = r * _NW + wid
        pltpu.sync_copy(cent_hbm.at[comp], row_v)
        for fc in range(_NFCH):
            pltpu.sync_copy(feat_hbm.at[comp, pl.ds(fc * _FCH, _FCH)], feat_v)

            def group_step(j, accs, fc=fc):
                out = []
                for k in range(4):
                    idx = lab_v[pl.ds(fc * _FCH + j * 4 * _LANES + k * _LANES,
                                      _LANES)]
                    g = plsc.load_gather(row_v, [idx])
                    f = feat_v[pl.ds(j * 4 * _LANES + k * _LANES, _LANES)]
                    d = f - g
                    out.append(accs[k] + d * d)
                return tuple(out)

            accs = plsc.parallel_loop(
                0, _FCH // (4 * _LANES), carry=accs, unroll=2)(group_step)

    acc = (accs[0] + accs[1]) + (accs[2] + accs[3])
    acc_v[...] = acc * (1.0 / _B)
    pltpu.sync_copy(acc_v, out_hbm.at[wid])


@jax.jit
def kernel(features, labels, centers):
    labels_i = labels.astype(jnp.int32)
    feat_t = features.T
    cent_t = centers.T
    mesh = plsc.VectorSubcoreMesh(core_axis_name="c", subcore_axis_name="s")
    partials = pl.kernel(
        _cl_body,
        out_type=jax.ShapeDtypeStruct((_NW, _LANES), jnp.float32),
        mesh=mesh,
        scratch_types=[
            pltpu.VMEM((_V,), jnp.float32),
            pltpu.VMEM((_FCH,), jnp.float32),
            pltpu.VMEM((_B,), jnp.int32),
            pltpu.VMEM((_LANES,), jnp.float32),
            pltpu.SemaphoreType.DMA,
        ],
        compiler_params=pltpu.CompilerParams(use_tc_tiling_on_sc=True,
                                            needs_layout_passes=False),
    )(feat_t, labels_i, cent_t)
    return jnp.sum(partials)


# speedup vs baseline: 2.1217x; 1.0334x over previous
import functools

import jax
import jax.numpy as jnp
from jax import lax
from jax.experimental import pallas as pl
from jax.experimental.pallas import tpu as pltpu
from jax.experimental.pallas import tpu_sc as plsc

_B = 16384
_D = 64
_V = 100000
_NC = 2
_NS = 16
_NW = _NC * _NS
_ROUNDS = _D // _NW
_LANES = 16
_HALF = 50048
_HALF1 = _V - _HALF
_H1MAIN = 49920
_TAIL = _V - _HALF - _H1MAIN
_FCH = 4096
_NFCH = _B // _FCH
_TASKS = tuple((r, h) for r in range(_ROUNDS) for h in range(2))


def _cl_body(feat_hbm, lab_hbm, cent_hbm, tail_hbm, out_hbm, row_a, row_b,
             feat_a,
             feat_b, lab_v, acc_v, sem_ra, sem_rb, sem_fa, sem_fb, sem_l):
    c = lax.axis_index("c")
    s = lax.axis_index("s")
    wid = s * _NC + c

    rows = (row_a, row_b)
    row_sems = (sem_ra, sem_rb)
    feats = (feat_a, feat_b)
    feat_sems = (sem_fa, sem_fb)

    def comp(r):
        return r * _NW + wid

    def row_start(t):
        r, h = _TASKS[t]
        if h == 0:
            return (pltpu.async_copy(
                cent_hbm.at[comp(r), pl.ds(0, _HALF)],
                rows[t % 2].at[pl.ds(0, _HALF)], row_sems[t % 2]),)
        return (
            pltpu.async_copy(
                cent_hbm.at[comp(r), pl.ds(_HALF, _H1MAIN)],
                rows[t % 2].at[pl.ds(0, _H1MAIN)], row_sems[t % 2]),
            pltpu.async_copy(
                tail_hbm.at[comp(r)],
                rows[t % 2].at[pl.ds(_H1MAIN, 128)], row_sems[t % 2]),
        )

    def feat_start(q):
        t, fc = divmod(q, _NFCH)
        r, _ = _TASKS[t]
        return pltpu.async_copy(
            feat_hbm.at[comp(r), pl.ds(fc * _FCH, _FCH)],
            feats[q % 2], feat_sems[q % 2])

    lab_cp = pltpu.async_copy(lab_hbm, lab_v, sem_l)
    row_cp = row_start(0)
    feat_cp = feat_start(0)
    lab_cp.wait()

    zero = jnp.zeros((_LANES,), jnp.float32)
    accs = (zero, zero, zero, zero)
    for t, (r, h) in enumerate(_TASKS):
        next_row_cp = row_start(t + 1) if t + 1 < len(_TASKS) else None
        for cp in row_cp:
            cp.wait()
        row_v = rows[t % 2]
        for fc in range(_NFCH):
            q = t * _NFCH + fc
            next_feat_cp = feat_start(q + 1) if q + 1 < len(_TASKS) * _NFCH \
                else None
            feat_cp.wait()
            feat_v = feats[q % 2]

            def group_step(j, accs, fc=fc, h=h, row_v=row_v, feat_v=feat_v):
                out = []
                for k in range(4):
                    off = fc * _FCH + (j * 4 + k) * _LANES
                    idx = lab_v[pl.ds(off, _LANES)]
                    if h == 0:
                        idxl = jnp.minimum(idx, _HALF - 1)
                        m = idx < _HALF
                    else:
                        sh = idx - _HALF
                        idxl = jnp.maximum(sh, 0)
                        m = idx >= _HALF
                    g = plsc.load_gather(row_v, [idxl])
                    f = feat_v[pl.ds((j * 4 + k) * _LANES, _LANES)]
                    d = f - g
                    out.append(accs[k] + jnp.where(m, d * d, 0.0))
                return tuple(out)

            accs = plsc.parallel_loop(
                0, _FCH // (4 * _LANES), carry=accs, unroll=2)(group_step)
            feat_cp = next_feat_cp
        row_cp = next_row_cp

    acc = (accs[0] + accs[1]) + (accs[2] + accs[3])
    acc_v[...] = acc * (1.0 / _B)
    pltpu.sync_copy(acc_v, out_hbm.at[wid])


@jax.jit
def kernel(features, labels, centers):
    labels_i = labels.astype(jnp.int32)
    feat_t = features.T
    cent_t = centers.T
    tail_t = jnp.pad(cent_t[:, _HALF + _H1MAIN:],
                     ((0, 0), (0, 128 - _TAIL)))
    mesh = plsc.VectorSubcoreMesh(core_axis_name="c", subcore_axis_name="s")
    partials = pl.kernel(
        _cl_body,
        out_type=jax.ShapeDtypeStruct((_NW, _LANES), jnp.float32),
        mesh=mesh,
        scratch_types=[
            pltpu.VMEM((_HALF,), jnp.float32),
            pltpu.VMEM((_HALF,), jnp.float32),
            pltpu.VMEM((_FCH,), jnp.float32),
            pltpu.VMEM((_FCH,), jnp.float32),
            pltpu.VMEM((_B,), jnp.int32),
            pltpu.VMEM((_LANES,), jnp.float32),
            pltpu.SemaphoreType.DMA,
            pltpu.SemaphoreType.DMA,
            pltpu.SemaphoreType.DMA,
            pltpu.SemaphoreType.DMA,
            pltpu.SemaphoreType.DMA,
        ],
        compiler_params=pltpu.CompilerParams(use_tc_tiling_on_sc=True,
                                             needs_layout_passes=False),
    )(feat_t, labels_i, cent_t, tail_t)
    return jnp.sum(partials)


# speedup vs baseline: 2.1260x; 1.0020x over previous
import functools

import jax
import jax.numpy as jnp
from jax import lax
from jax.experimental import pallas as pl
from jax.experimental.pallas import tpu as pltpu
from jax.experimental.pallas import tpu_sc as plsc

_B = 16384
_D = 64
_V = 100000
_NC = 2
_NS = 16
_NW = _NC * _NS
_ROUNDS = _D // _NW
_LANES = 16
_HALF = 49920
_HALF1 = _V - _HALF
_FCH = 4096
_NFCH = _B // _FCH
_TASKS = tuple((r, h) for r in range(_ROUNDS) for h in range(2))


def _cl_body(feat_hbm, lab_hbm, cent_hbm, out_hbm, row_a, row_b,
             feat_a,
             feat_b, lab_v, acc_v, sem_ra, sem_rb, sem_fa, sem_fb, sem_l):
    c = lax.axis_index("c")
    s = lax.axis_index("s")
    wid = s * _NC + c

    rows = (row_a, row_b)
    row_sems = (sem_ra, sem_rb)
    feats = (feat_a, feat_b)
    feat_sems = (sem_fa, sem_fb)

    def comp(r):
        return r * _NW + wid

    def row_start(t):
        r, h = _TASKS[t]
        size = _HALF if h == 0 else _HALF1
        return (pltpu.async_copy(
            cent_hbm.at[comp(r), pl.ds(h * _HALF, size)],
            rows[t % 2].at[pl.ds(0, size)], row_sems[t % 2]),)

    def feat_start(q):
        t, fc = divmod(q, _NFCH)
        r, _ = _TASKS[t]
        return pltpu.async_copy(
            feat_hbm.at[comp(r), pl.ds(fc * _FCH, _FCH)],
            feats[q % 2], feat_sems[q % 2])

    lab_cp = pltpu.async_copy(lab_hbm, lab_v, sem_l)
    row_cp = row_start(0)
    feat_cp = feat_start(0)
    lab_cp.wait()

    zero = jnp.zeros((_LANES,), jnp.float32)
    accs = (zero, zero, zero, zero)
    for t, (r, h) in enumerate(_TASKS):
        next_row_cp = row_start(t + 1) if t + 1 < len(_TASKS) else None
        for cp in row_cp:
            cp.wait()
        row_v = rows[t % 2]
        for fc in range(_NFCH):
            q = t * _NFCH + fc
            next_feat_cp = feat_start(q + 1) if q + 1 < len(_TASKS) * _NFCH \
                else None
            feat_cp.wait()
            feat_v = feats[q % 2]

            def group_step(j, accs, fc=fc, h=h, row_v=row_v, feat_v=feat_v):
                out = []
                for k in range(4):
                    off = fc * _FCH + (j * 4 + k) * _LANES
                    idx = lab_v[pl.ds(off, _LANES)]
                    if h == 0:
                        idxl = jnp.minimum(idx, _HALF - 1)
                        m = idx < _HALF
                    else:
                        sh = idx - _HALF
                        idxl = jnp.maximum(sh, 0)
                        m = idx >= _HALF
                    g = plsc.load_gather(row_v, [idxl])
                    f = feat_v[pl.ds((j * 4 + k) * _LANES, _LANES)]
                    d = f - g
                    out.append(accs[k] + jnp.where(m, d * d, 0.0))
                return tuple(out)

            accs = plsc.parallel_loop(
                0, _FCH // (4 * _LANES), carry=accs, unroll=2)(group_step)
            feat_cp = next_feat_cp
        row_cp = next_row_cp

    acc = (accs[0] + accs[1]) + (accs[2] + accs[3])
    acc_v[...] = acc * (1.0 / _B)
    pltpu.sync_copy(acc_v, out_hbm.at[wid])


@jax.jit
def kernel(features, labels, centers):
    labels_i = labels.astype(jnp.int32)
    feat_t = features.T
    cent_t = centers.T
    mesh = plsc.VectorSubcoreMesh(core_axis_name="c", subcore_axis_name="s")
    partials = pl.kernel(
        _cl_body,
        out_type=jax.ShapeDtypeStruct((_NW, _LANES), jnp.float32),
        mesh=mesh,
        scratch_types=[
            pltpu.VMEM((_HALF1,), jnp.float32),
            pltpu.VMEM((_HALF1,), jnp.float32),
            pltpu.VMEM((_FCH,), jnp.float32),
            pltpu.VMEM((_FCH,), jnp.float32),
            pltpu.VMEM((_B,), jnp.int32),
            pltpu.VMEM((_LANES,), jnp.float32),
            pltpu.SemaphoreType.DMA,
            pltpu.SemaphoreType.DMA,
            pltpu.SemaphoreType.DMA,
            pltpu.SemaphoreType.DMA,
            pltpu.SemaphoreType.DMA,
        ],
        compiler_params=pltpu.CompilerParams(use_tc_tiling_on_sc=True,
                                             needs_layout_passes=False),
    )(feat_t, labels_i, cent_t)
    return jnp.sum(partials)


# speedup vs baseline: 2.3934x; 1.1258x over previous
import functools

import jax
import jax.numpy as jnp
from jax import lax
from jax.experimental import pallas as pl
from jax.experimental.pallas import tpu as pltpu
from jax.experimental.pallas import tpu_sc as plsc

_B = 16384
_D = 64
_V = 100000
_NC = 2
_NS = 16
_NW = _NC * _NS
_ROUNDS = _D // _NW
_LANES = 16
_FCH = 4096
_NFCH = _B // _FCH


def _cl_body(feat_hbm, lab_hbm, cent_hbm, out_hbm, row_v, feat_a, feat_b,
             lab_v, acc_v, sem_r, sem_fa, sem_fb, sem_l):
    c = lax.axis_index("c")
    s = lax.axis_index("s")
    wid = s * _NC + c

    feats = (feat_a, feat_b)
    feat_sems = (sem_fa, sem_fb)

    def comp(r):
        return r * _NW + wid

    def feat_start(q):
        r, fc = divmod(q, _NFCH)
        return pltpu.async_copy(
            feat_hbm.at[comp(r), pl.ds(fc * _FCH, _FCH)],
            feats[q % 2], feat_sems[q % 2])

    lab_cp = pltpu.async_copy(lab_hbm, lab_v, sem_l)
    row_cp = pltpu.async_copy(cent_hbm.at[comp(0)], row_v, sem_r)
    feat_cp = feat_start(0)
    lab_cp.wait()

    zero = jnp.zeros((_LANES,), jnp.float32)
    accs = (zero, zero, zero, zero)
    for r in range(_ROUNDS):
        row_cp.wait()
        for fc in range(_NFCH):
            q = r * _NFCH + fc
            next_feat_cp = feat_start(q + 1) \
                if q + 1 < _ROUNDS * _NFCH else None
            feat_cp.wait()
            feat_v = feats[q % 2]

            def group_step(j, accs, fc=fc, feat_v=feat_v):
                out = []
                for k in range(4):
                    off = fc * _FCH + (j * 4 + k) * _LANES
                    idx = lab_v[pl.ds(off, _LANES)]
                    g = plsc.load_gather(row_v, [idx])
                    f = feat_v[pl.ds((j * 4 + k) * _LANES, _LANES)]
                    d = f - g
                    out.append(accs[k] + d * d)
                return tuple(out)

            accs = plsc.parallel_loop(
                0, _FCH // (4 * _LANES), carry=accs, unroll=2)(group_step)
            feat_cp = next_feat_cp
        if r + 1 < _ROUNDS:
            row_cp = pltpu.async_copy(cent_hbm.at[comp(r + 1)], row_v, sem_r)

    acc = (accs[0] + accs[1]) + (accs[2] + accs[3])
    acc_v[...] = acc * (1.0 / _B)
    pltpu.sync_copy(acc_v, out_hbm.at[wid])


@jax.jit
def kernel(features, labels, centers):
    labels_i = labels.astype(jnp.int32)
    feat_t = features.T
    cent_t = centers.T
    mesh = plsc.VectorSubcoreMesh(core_axis_name="c", subcore_axis_name="s")
    partials = pl.kernel(
        _cl_body,
        out_type=jax.ShapeDtypeStruct((_NW, _LANES), jnp.float32),
        mesh=mesh,
        scratch_types=[
            pltpu.VMEM((_V,), jnp.float32),
            pltpu.VMEM((_FCH,), jnp.float32),
            pltpu.VMEM((_FCH,), jnp.float32),
            pltpu.VMEM((_B,), jnp.int32),
            pltpu.VMEM((_LANES,), jnp.float32),
            pltpu.SemaphoreType.DMA,
            pltpu.SemaphoreType.DMA,
            pltpu.SemaphoreType.DMA,
            pltpu.SemaphoreType.DMA,
        ],
        compiler_params=pltpu.CompilerParams(use_tc_tiling_on_sc=True,
                                             needs_layout_passes=False),
    )(feat_t, labels_i, cent_t)
    return jnp.sum(partials)


# speedup vs baseline: 2.3951x; 1.0007x over previous
import functools

import jax
import jax.numpy as jnp
from jax import lax
from jax.experimental import pallas as pl
from jax.experimental.pallas import tpu as pltpu
from jax.experimental.pallas import tpu_sc as plsc

_B = 16384
_D = 64
_V = 100000
_NC = 2
_NS = 16
_NW = _NC * _NS
_ROUNDS = _D // _NW
_LANES = 16
_FCH = 4096
_NFCH = _B // _FCH


def _cl_body(feat_hbm, lab_hbm, cent_hbm, out_hbm, row_v, feat_a, feat_b,
             lab_v, acc_v, sem_r, sem_fa, sem_fb, sem_l):
    c = lax.axis_index("c")
    s = lax.axis_index("s")
    wid = s * _NC + c

    feats = (feat_a, feat_b)
    feat_sems = (sem_fa, sem_fb)

    def comp(r):
        return r * _NW + wid

    def feat_start(q):
        r, fc = divmod(q, _NFCH)
        return pltpu.async_copy(
            feat_hbm.at[comp(r), pl.ds(fc * _FCH, _FCH)],
            feats[q % 2], feat_sems[q % 2])

    def row_start(r):
        return pltpu.async_copy(cent_hbm.at[comp(r)], row_v, sem_r)

    lab_cp = pltpu.async_copy(lab_hbm, lab_v, sem_l)
    row_cp = row_start(0)
    feat_cp = feat_start(0)
    lab_cp.wait()

    zero = jnp.zeros((_LANES,), jnp.float32)
    accs = (zero, zero, zero, zero)
    for r in range(_ROUNDS):
        row_cp.wait()
        for fc in range(_NFCH):
            q = r * _NFCH + fc
            next_feat_cp = feat_start(q + 1) \
                if q + 1 < _ROUNDS * _NFCH else None
            feat_cp.wait()
            feat_v = feats[q % 2]

            def group_step(j, accs, fc=fc, feat_v=feat_v):
                out = []
                for k in range(4):
                    off = fc * _FCH + (j * 4 + k) * _LANES
                    idx = lab_v[pl.ds(off, _LANES)]
                    g = plsc.load_gather(row_v, [idx])
                    f = feat_v[pl.ds((j * 4 + k) * _LANES, _LANES)]
                    d = f - g
                    out.append(accs[k] + d * d)
                return tuple(out)

            accs = plsc.parallel_loop(
                0, _FCH // (4 * _LANES), carry=accs, unroll=2)(group_step)
            feat_cp = next_feat_cp
        if r + 1 < _ROUNDS:
            row_cp = row_start(r + 1)

    acc = (accs[0] + accs[1]) + (accs[2] + accs[3])
    acc_v[...] = acc * (1.0 / _B)
    pltpu.sync_copy(acc_v, out_hbm.at[wid])


@jax.jit
def kernel(features, labels, centers):
    labels_i = labels.astype(jnp.int32)
    feat_t = features.T
    cent_t = centers.T
    mesh = plsc.VectorSubcoreMesh(core_axis_name="c", subcore_axis_name="s")
    partials = pl.kernel(
        _cl_body,
        out_type=jax.ShapeDtypeStruct((_NW, _LANES), jnp.float32),
        mesh=mesh,
        scratch_types=[
            pltpu.VMEM((_V,), jnp.float32),
            pltpu.VMEM((_FCH,), jnp.float32),
            pltpu.VMEM((_FCH,), jnp.float32),
            pltpu.VMEM((_B,), jnp.int32),
            pltpu.VMEM((_LANES,), jnp.float32),
            pltpu.SemaphoreType.DMA,
            pltpu.SemaphoreType.DMA,
            pltpu.SemaphoreType.DMA,
            pltpu.SemaphoreType.DMA,
        ],
        compiler_params=pltpu.CompilerParams(use_tc_tiling_on_sc=True,
                                             needs_layout_passes=False),
    )(feat_t, labels_i, cent_t)
    return jnp.sum(partials)
